# baseline (speedup 1.0000x reference)
import functools

import jax
import jax.numpy as jnp
from jax import lax
from jax.experimental import pallas as pl
from jax.experimental.pallas import tpu as pltpu
from jax.experimental.pallas import tpu_sc as plsc

_BATCH = 16384
_FEAT = 64
_NC = 2
_NS = 16
_NW = _NC * _NS
_BPW = _BATCH // _NW
_IDX_CHUNK = 128
_NCHUNK = _BPW // _IDX_CHUNK


def _body(emb_hbm, lab_hbm, cen_hbm, out_hbm, idx_v, cen_v, emb_v, acc_v,
          gsem, esem):
    wid = lax.axis_index("s") * _NC + lax.axis_index("c")
    base = wid * _BPW

    pltpu.sync_copy(lab_hbm.at[wid], idx_v)

    ecopy = pltpu.async_copy(emb_hbm.at[pl.ds(base, _BPW)], emb_v, esem)
    gathers = [
        pltpu.async_copy(
            cen_hbm.at[idx_v.at[j]],
            cen_v.at[pl.ds(j * _IDX_CHUNK, _IDX_CHUNK)],
            gsem,
        )
        for j in range(_NCHUNK)
    ]
    ecopy.wait()
    for g in gathers:
        g.wait()

    def step(i, acc):
        for c in range(_FEAT // 16):
            e = emb_v[i, pl.ds(c * 16, 16)]
            g = cen_v[i, pl.ds(c * 16, 16)]
            d = e - g
            acc = acc + d * d
        return acc

    acc = lax.fori_loop(0, _BPW, step, jnp.zeros((16,), jnp.float32))
    acc_v[...] = acc * (1.0 / (2.0 * _BATCH))
    pltpu.sync_copy(acc_v, out_hbm.at[wid])


@jax.jit
def _center_loss(embeddings, labels, centers):
    lab = labels.astype(jnp.int32).reshape(_NW, _NCHUNK, _IDX_CHUNK)
    kern = pl.kernel(
        _body,
        out_type=jax.ShapeDtypeStruct((_NW, 16), jnp.float32),
        mesh=plsc.VectorSubcoreMesh(core_axis_name="c", subcore_axis_name="s"),
        scratch_types=[
            pltpu.VMEM((_NCHUNK, _IDX_CHUNK), jnp.int32),
            pltpu.VMEM((_BPW, _FEAT), jnp.float32),
            pltpu.VMEM((_BPW, _FEAT), jnp.float32),
            pltpu.VMEM((16,), jnp.float32),
            pltpu.SemaphoreType.DMA,
            pltpu.SemaphoreType.DMA,
        ],
        compiler_params=pltpu.CompilerParams(use_tc_tiling_on_sc=False),
    )
    partials = kern(embeddings, lab, centers)
    return jnp.sum(partials)


def kernel(embeddings, labels, centers):
    return _center_loss(embeddings, labels, centers)
